# Initial kernel scaffold; baseline (speedup 1.0000x reference)
import jax
import jax.numpy as jnp
from jax.experimental import pallas as pl


def kernel(idx, targets, table):
    raise NotImplementedError("write your pallas kernel here")


# speedup vs baseline: 1.3045x; 1.3045x over previous
import functools

import jax
import jax.numpy as jnp
from jax import lax
from jax.experimental import pallas as pl
from jax.experimental.pallas import tpu as pltpu
from jax.experimental.pallas import tpu_sc as plsc

LANES = 16


def _lse_body(table_ref, lse_ref):
    x = table_ref[...]
    m = jnp.max(x, axis=1)
    s = jnp.sum(jnp.exp(x - m[:, None]), axis=1)
    lse_ref[...] = m + jnp.log(s)


def _row_lse(table):
    v = table.shape[0]
    return pl.pallas_call(
        _lse_body,
        out_shape=jax.ShapeDtypeStruct((v,), jnp.float32),
    )(table)


def _make_sc_call(tok, v, nw, chunk):
    per_w = tok // nw
    nchunk = per_w // chunk
    mesh = plsc.VectorSubcoreMesh(core_axis_name="c", subcore_axis_name="s")

    @functools.partial(
        pl.kernel,
        mesh=mesh,
        out_type=[
            jax.ShapeDtypeStruct((tok, v), jnp.float32),
            jax.ShapeDtypeStruct((nw, LANES), jnp.float32),
        ],
        scratch_types=[
            pltpu.VMEM((per_w,), jnp.int32),
            pltpu.VMEM((per_w,), jnp.int32),
            pltpu.VMEM((v,), jnp.float32),
            pltpu.VMEM((chunk, v), jnp.float32),
            pltpu.VMEM((LANES,), jnp.float32),
            pltpu.SemaphoreType.DMA,
        ],
        compiler_params=pltpu.CompilerParams(
            use_tc_tiling_on_sc=False, needs_layout_passes=False),
    )
    def sc_call(idx_hbm, tgt_hbm, lse_hbm, table_hbm, out_hbm,
                part_hbm, idx_v, tgt_v, lse_v, rows_v, acc_v, sem):
        cid = lax.axis_index("c")
        sid = lax.axis_index("s")
        wid = sid * 2 + cid
        base = wid * per_w

        pltpu.sync_copy(idx_hbm.at[pl.ds(base, per_w)], idx_v)
        pltpu.sync_copy(tgt_hbm.at[pl.ds(base, per_w)], tgt_v)
        pltpu.sync_copy(lse_hbm, lse_v)

        def chunk_body(ci, acc):
            off = ci * chunk
            pltpu.async_copy(
                table_hbm.at[idx_v.at[pl.ds(off, chunk)]], rows_v, sem
            ).wait()

            def vbody(j, a):
                rid = lax.iota(jnp.int32, LANES) + j * LANES
                iv = idx_v[pl.ds(off + j * LANES, LANES)]
                tv = tgt_v[pl.ds(off + j * LANES, LANES)]
                vals = plsc.load_gather(rows_v, [rid, tv])
                lg = plsc.load_gather(lse_v, [iv])
                return a + (lg - vals)

            acc = lax.fori_loop(0, chunk // LANES, vbody, acc)
            pltpu.sync_copy(rows_v, out_hbm.at[pl.ds(base + off, chunk)])
            return acc

        acc = lax.fori_loop(
            0, nchunk, chunk_body, jnp.zeros((LANES,), jnp.float32)
        )
        acc_v[...] = acc
        pltpu.sync_copy(acc_v, part_hbm.at[wid])

    return sc_call


def kernel(idx, targets, table):
    b, t = idx.shape
    v = table.shape[0]
    tok = b * t
    nw = 32
    chunk = 64

    idx_f = idx.reshape(tok)
    tgt_f = targets.reshape(tok)
    lse = _row_lse(table)
    out, parts = _make_sc_call(tok, v, nw, chunk)(idx_f, tgt_f, lse, table)
    logits = out.reshape(b, t, v)
    loss = jnp.sum(parts) / tok
    return (logits, loss)


# speedup vs baseline: 1.3376x; 1.0254x over previous
import functools

import jax
import jax.numpy as jnp
from jax import lax
from jax.experimental import pallas as pl
from jax.experimental.pallas import tpu as pltpu
from jax.experimental.pallas import tpu_sc as plsc

LANES = 16


def _lse_body(table_ref, lse_ref):
    x = table_ref[...]
    m = jnp.max(x, axis=1)
    s = jnp.sum(jnp.exp(x - m[:, None]), axis=1)
    lse_ref[...] = m + jnp.log(s)


def _row_lse(table):
    v = table.shape[0]
    return pl.pallas_call(
        _lse_body,
        out_shape=jax.ShapeDtypeStruct((v,), jnp.float32),
    )(table)


def _make_sc_call(tok, v, nw, chunk):
    per_w = tok // nw
    nchunk = per_w // chunk
    mesh = plsc.VectorSubcoreMesh(core_axis_name="c", subcore_axis_name="s")

    @functools.partial(
        pl.kernel,
        mesh=mesh,
        out_type=[
            jax.ShapeDtypeStruct((tok, v), jnp.float32),
            jax.ShapeDtypeStruct((nw, LANES), jnp.float32),
        ],
        scratch_types=[
            pltpu.VMEM((per_w,), jnp.int32),
            pltpu.VMEM((per_w,), jnp.int32),
            pltpu.VMEM((v,), jnp.float32),
            pltpu.VMEM((chunk, v), jnp.float32),
            pltpu.VMEM((chunk, v), jnp.float32),
            pltpu.VMEM((LANES,), jnp.float32),
            pltpu.SemaphoreType.DMA,
            pltpu.SemaphoreType.DMA,
            pltpu.SemaphoreType.DMA,
            pltpu.SemaphoreType.DMA,
        ],
        compiler_params=pltpu.CompilerParams(
            use_tc_tiling_on_sc=False, needs_layout_passes=False),
    )
    def sc_call(idx_hbm, tgt_hbm, lse_hbm, table_hbm, out_hbm,
                part_hbm, idx_v, tgt_v, lse_v, rows0_v, rows1_v, acc_v,
                gsem0, gsem1, ssem0, ssem1):
        cid = lax.axis_index("c")
        sid = lax.axis_index("s")
        wid = sid * 2 + cid
        base = wid * per_w

        pltpu.sync_copy(idx_hbm.at[pl.ds(base, per_w)], idx_v)
        pltpu.sync_copy(tgt_hbm.at[pl.ds(base, per_w)], tgt_v)
        pltpu.sync_copy(lse_hbm, lse_v)

        bufs = ((rows0_v, gsem0, ssem0), (rows1_v, gsem1, ssem1))

        def gather(ci, rows_v, gsem):
            return pltpu.make_async_copy(
                table_hbm.at[idx_v.at[pl.ds(ci * chunk, chunk)]], rows_v,
                gsem)

        def scatter(ci, rows_v, ssem):
            return pltpu.make_async_copy(
                rows_v, out_hbm.at[pl.ds(base + ci * chunk, chunk)], ssem)

        gather(0, rows0_v, gsem0).start()
        gather(1, rows1_v, gsem1).start()

        def pair_body(g, acc):
            for b, (rows_v, gsem, ssem) in enumerate(bufs):
                ci = 2 * g + b
                gather(ci, rows_v, gsem).wait()
                scatter(ci, rows_v, ssem).start()

                def vbody(j, a):
                    rid = lax.iota(jnp.int32, LANES) + j * LANES
                    iv = idx_v[pl.ds(ci * chunk + j * LANES, LANES)]
                    tv = tgt_v[pl.ds(ci * chunk + j * LANES, LANES)]
                    vals = plsc.load_gather(rows_v, [rid, tv])
                    lg = plsc.load_gather(lse_v, [iv])
                    return a + (lg - vals)

                acc = lax.fori_loop(0, chunk // LANES, vbody, acc)

                @pl.when(ci + 2 < nchunk)
                def _():
                    scatter(ci, rows_v, ssem).wait()
                    gather(ci + 2, rows_v, gsem).start()

            return acc

        acc = lax.fori_loop(
            0, nchunk // 2, pair_body, jnp.zeros((LANES,), jnp.float32)
        )
        scatter(nchunk - 2, rows0_v, ssem0).wait()
        scatter(nchunk - 1, rows1_v, ssem1).wait()
        acc_v[...] = acc
        pltpu.sync_copy(acc_v, part_hbm.at[wid])

    return sc_call


def kernel(idx, targets, table):
    b, t = idx.shape
    v = table.shape[0]
    tok = b * t
    nw = 32
    chunk = 32

    idx_f = idx.reshape(tok)
    tgt_f = targets.reshape(tok)
    lse = _row_lse(table)
    out, parts = _make_sc_call(tok, v, nw, chunk)(idx_f, tgt_f, lse, table)
    logits = out.reshape(b, t, v)
    loss = jnp.sum(parts) / tok
    return (logits, loss)


# speedup vs baseline: 1.5198x; 1.1362x over previous
import functools

import jax
import jax.numpy as jnp
from jax import lax
from jax.experimental import pallas as pl
from jax.experimental.pallas import tpu as pltpu
from jax.experimental.pallas import tpu_sc as plsc

LANES = 16


def _lse_body(table_ref, lse_ref):
    x = table_ref[...]
    m = jnp.max(x, axis=1)
    s = jnp.sum(jnp.exp(x - m[:, None]), axis=1)
    lse_ref[...] = m + jnp.log(s)


def _row_lse(table):
    v = table.shape[0]
    return pl.pallas_call(
        _lse_body,
        out_shape=jax.ShapeDtypeStruct((v,), jnp.float32),
    )(table)


def _make_sc_call(tok, v, nw, chunk, nbuf):
    per_w = tok // nw
    nchunk = per_w // chunk
    assert nbuf >= 3
    mesh = plsc.VectorSubcoreMesh(core_axis_name="c", subcore_axis_name="s")

    @functools.partial(
        pl.kernel,
        mesh=mesh,
        out_type=[
            jax.ShapeDtypeStruct((tok, v), jnp.float32),
            jax.ShapeDtypeStruct((nw, LANES), jnp.float32),
        ],
        scratch_types=[
            pltpu.VMEM((per_w,), jnp.int32),
            pltpu.VMEM((per_w,), jnp.int32),
            pltpu.VMEM((v,), jnp.float32),
            pltpu.VMEM_SHARED((v, v), jnp.float32),
            [pltpu.VMEM((chunk, v), jnp.float32) for _ in range(nbuf)],
            pltpu.VMEM((LANES,), jnp.float32),
            [pltpu.SemaphoreType.DMA for _ in range(nbuf)],
            [pltpu.SemaphoreType.DMA for _ in range(nbuf)],
        ],
        compiler_params=pltpu.CompilerParams(
            use_tc_tiling_on_sc=False, needs_layout_passes=False),
    )
    def sc_call(idx_hbm, tgt_hbm, lse_hbm, table_hbm, out_hbm,
                part_hbm, idx_v, tgt_v, lse_v, table_sh, rows_bufs, acc_v,
                gsems, ssems):
        cid = lax.axis_index("c")
        sid = lax.axis_index("s")
        wid = sid * 2 + cid
        base = wid * per_w

        @pl.when(sid == 0)
        def _():
            pltpu.sync_copy(table_hbm, table_sh)

        pltpu.sync_copy(idx_hbm.at[pl.ds(base, per_w)], idx_v)
        pltpu.sync_copy(tgt_hbm.at[pl.ds(base, per_w)], tgt_v)
        pltpu.sync_copy(lse_hbm, lse_v)
        plsc.subcore_barrier()

        def gather(ci, b):
            return pltpu.make_async_copy(
                table_sh.at[idx_v.at[pl.ds(ci * chunk, chunk)]],
                rows_bufs[b], gsems[b])

        def scatter(ci, b):
            return pltpu.make_async_copy(
                rows_bufs[b], out_hbm.at[pl.ds(base + ci * chunk, chunk)],
                ssems[b])

        gather(0, 0).start()

        def chunk_step(ci, b, acc):
            gather(ci, b).wait()
            scatter(ci, b).start()

            def vbody(j, a):
                rid = lax.iota(jnp.int32, LANES) + j * LANES
                iv = idx_v[pl.ds(ci * chunk + j * LANES, LANES)]
                tv = tgt_v[pl.ds(ci * chunk + j * LANES, LANES)]
                vals = plsc.load_gather(rows_bufs[b], [rid, tv])
                lg = plsc.load_gather(lse_v, [iv])
                return a + (lg - vals)

            acc = lax.fori_loop(0, chunk // LANES, vbody, acc)

            nb = (b + 1) % nbuf

            @pl.when(ci + 1 < nchunk)
            def _():
                @pl.when(ci + 1 >= nbuf)
                def _():
                    scatter(ci + 1 - nbuf, nb).wait()

                gather(ci + 1, nb).start()

            return acc

        def ring_body(g, acc):
            for b in range(nbuf):
                acc = chunk_step(g * nbuf + b, b, acc)
            return acc

        rounds = nchunk // nbuf
        acc = lax.fori_loop(
            0, rounds, ring_body, jnp.zeros((LANES,), jnp.float32)
        )
        for b in range(nchunk % nbuf):
            acc = chunk_step(rounds * nbuf + b, b, acc)
        for j in range(nchunk - nbuf, nchunk):
            scatter(j, j % nbuf).wait()
        acc_v[...] = acc
        pltpu.sync_copy(acc_v, part_hbm.at[wid])

    return sc_call


def kernel(idx, targets, table):
    b, t = idx.shape
    v = table.shape[0]
    tok = b * t
    nw = 32
    chunk = 16
    nbuf = 3

    idx_f = idx.reshape(tok)
    tgt_f = targets.reshape(tok)
    lse = _row_lse(table)
    out, parts = _make_sc_call(tok, v, nw, chunk, nbuf)(
        idx_f, tgt_f, lse, table)
    logits = out.reshape(b, t, v)
    loss = jnp.sum(parts) / tok
    return (logits, loss)


# speedup vs baseline: 2.5816x; 1.6987x over previous
import functools

import jax
import jax.numpy as jnp
from jax import lax
from jax.experimental import pallas as pl
from jax.experimental.pallas import tpu as pltpu
from jax.experimental.pallas import tpu_sc as plsc

LANES = 16


def _lse_body(table_ref, lse_ref, padt_ref):
    x = table_ref[...]
    m = jnp.max(x, axis=1)
    s = jnp.sum(jnp.exp(x - m[:, None]), axis=1)
    lse_ref[...] = m + jnp.log(s)
    v = x.shape[1]
    padt_ref[:, :v] = x
    padt_ref[:, v:] = jnp.zeros_like(padt_ref[:, v:])


def _row_lse(table):
    v = table.shape[0]
    vp = ((table.shape[1] + 127) // 128) * 128
    return pl.pallas_call(
        _lse_body,
        out_shape=[
            jax.ShapeDtypeStruct((v,), jnp.float32),
            jax.ShapeDtypeStruct((v, vp), jnp.float32),
        ],
    )(table)


def _make_sc_call(tok, v, nw, nbuf):
    per_w = tok // nw
    group = LANES
    ngrp = per_w // group
    nlt = (v + 127) // 128
    tail = v - (nlt - 1) * 128
    lst = nlt * group
    assert lst <= 128
    nrt = (v + 7) // 8
    mesh = plsc.VectorSubcoreMesh(core_axis_name="c", subcore_axis_name="s")

    @functools.partial(
        pl.kernel,
        mesh=mesh,
        out_type=[
            jax.ShapeDtypeStruct((tok, v), jnp.float32),
            jax.ShapeDtypeStruct((nw, LANES), jnp.float32),
            jax.ShapeDtypeStruct((tok, 128), jnp.float32),
        ],
        scratch_types=[
            pltpu.VMEM((per_w,), jnp.int32),
            pltpu.VMEM((per_w,), jnp.int32),
            pltpu.VMEM((v,), jnp.float32),
            pltpu.VMEM_SHARED((nlt * v, 128), jnp.float32),
            [pltpu.VMEM((lst, 128), jnp.float32) for _ in range(nbuf)],
            [pltpu.VMEM((lst,), jnp.int32) for _ in range(nbuf)],
            pltpu.VMEM((LANES,), jnp.float32),
            [pltpu.SemaphoreType.DMA for _ in range(nbuf)],
            [pltpu.SemaphoreType.DMA for _ in range(nbuf)],
        ],
        compiler_params=pltpu.CompilerParams(needs_layout_passes=False),
    )
    def sc_call(idx_hbm, tgt_hbm, lse_hbm, table_hbm, out_hbm,
                part_hbm, tail_hbm, idx_v, tgt_v, lse_v, table_sh,
                img_bufs, list_bufs, acc_v, gsems, ssems):
        cid = lax.axis_index("c")
        sid = lax.axis_index("s")
        wid = sid * 2 + cid
        base = wid * per_w

        rt_per = (nrt + LANES - 1) // LANES
        for j in range(rt_per):
            rt = sid * rt_per + j

            @pl.when(rt < nrt)
            def _():
                for lt in range(nlt):
                    pltpu.sync_copy(
                        table_hbm.at[pl.ds(rt * 8, 8),
                                     pl.ds(lt * 128, 128)],
                        table_sh.at[pl.ds(lt * v + rt * 8, 8)],
                    )

        pltpu.sync_copy(idx_hbm.at[pl.ds(base, per_w)], idx_v)
        pltpu.sync_copy(tgt_hbm.at[pl.ds(base, per_w)], tgt_v)
        pltpu.sync_copy(lse_hbm, lse_v)
        plsc.subcore_barrier()

        def build_list(g, b):
            iv = idx_v[pl.ds(g * group, LANES)]
            for lt in range(nlt):
                list_bufs[b][pl.ds(lt * LANES, LANES)] = iv + lt * v

        def gather(g, b):
            return pltpu.make_async_copy(
                table_sh.at[list_bufs[b]], img_bufs[b], gsems[b])

        def scatters(g, b):
            row0 = base + g * group
            cps = []
            for lt in range(nlt - 1):
                cps.append(pltpu.make_async_copy(
                    img_bufs[b].at[pl.ds(lt * group, group)],
                    out_hbm.at[pl.ds(row0, group), pl.ds(lt * 128, 128)],
                    ssems[b]))
            cps.append(pltpu.make_async_copy(
                img_bufs[b].at[pl.ds((nlt - 1) * group, group)],
                tail_hbm.at[pl.ds(row0, group)],
                ssems[b]))
            return cps

        build_list(0, 0)
        gather(0, 0).start()

        def group_step(g, b, acc):
            gather(g, b).wait()
            for cp in scatters(g, b):
                cp.start()

            iv = idx_v[pl.ds(g * group, LANES)]
            tv = tgt_v[pl.ds(g * group, LANES)]
            rows = (tv // 128) * group + lax.iota(jnp.int32, LANES)
            cols = tv % 128
            tval = plsc.load_gather(img_bufs[b], [rows, cols])
            lg = plsc.load_gather(lse_v, [iv])
            acc = acc + (lg - tval)

            nb = (b + 1) % nbuf

            @pl.when(g + 1 < ngrp)
            def _():
                @pl.when(g + 1 >= nbuf)
                def _():
                    for cp in scatters(g + 1 - nbuf, nb):
                        cp.wait()

                build_list(g + 1, nb)
                gather(g + 1, nb).start()

            return acc

        def ring_body(r, acc):
            for b in range(nbuf):
                acc = group_step(r * nbuf + b, b, acc)
            return acc

        rounds = ngrp // nbuf
        acc = lax.fori_loop(
            0, rounds, ring_body, jnp.zeros((LANES,), jnp.float32)
        )
        for b in range(ngrp % nbuf):
            acc = group_step(rounds * nbuf + b, b, acc)
        for g in range(ngrp - nbuf, ngrp):
            for cp in scatters(g, g % nbuf):
                cp.wait()
        acc_v[...] = acc
        pltpu.sync_copy(acc_v, part_hbm.at[wid])

    return sc_call


def kernel(idx, targets, table):
    b, t = idx.shape
    v = table.shape[0]
    tok = b * t
    nw = 32
    nbuf = 3

    idx_f = idx.reshape(tok)
    tgt_f = targets.reshape(tok)
    lse, padt = _row_lse(table)
    out, parts, tail_buf = _make_sc_call(tok, v, nw, nbuf)(
        idx_f, tgt_f, lse, padt)
    vfull = 128 * (v // 128)
    out = out.at[:, vfull:].set(tail_buf[:, : v - vfull])
    logits = out.reshape(b, t, v)
    loss = jnp.sum(parts) / tok
    return (logits, loss)


# speedup vs baseline: 2.7427x; 1.0624x over previous
import functools

import jax
import jax.numpy as jnp
from jax import lax
from jax.experimental import pallas as pl
from jax.experimental.pallas import tpu as pltpu
from jax.experimental.pallas import tpu_sc as plsc

LANES = 16


def _lse_body(table_ref, lse_ref, padt_ref):
    x = table_ref[...]
    m = jnp.max(x, axis=1)
    s = jnp.sum(jnp.exp(x - m[:, None]), axis=1)
    lse_ref[...] = m + jnp.log(s)
    v = x.shape[1]
    padt_ref[:, :v] = x
    padt_ref[:, v:] = jnp.zeros_like(padt_ref[:, v:])


def _row_lse(table):
    v = table.shape[0]
    vp = ((table.shape[1] + 127) // 128) * 128
    return pl.pallas_call(
        _lse_body,
        out_shape=[
            jax.ShapeDtypeStruct((v,), jnp.float32),
            jax.ShapeDtypeStruct((v, vp), jnp.float32),
        ],
    )(table)


def _make_sc_call(tok, v, nw, nbuf):
    per_w = tok // nw
    group = LANES
    ngrp = per_w // group
    nlt = (v + 127) // 128
    tail = v - (nlt - 1) * 128
    lst = nlt * group
    assert lst <= 128
    nrt = (v + 7) // 8
    mesh = plsc.VectorSubcoreMesh(core_axis_name="c", subcore_axis_name="s")

    @functools.partial(
        pl.kernel,
        mesh=mesh,
        out_type=[
            jax.ShapeDtypeStruct((tok // 8, 8, v), jnp.float32),
            jax.ShapeDtypeStruct((nw, LANES), jnp.float32),
        ],
        scratch_types=[
            pltpu.VMEM((per_w,), jnp.int32),
            pltpu.VMEM((per_w,), jnp.int32),
            pltpu.VMEM((v,), jnp.float32),
            pltpu.VMEM_SHARED((nlt * v, 128), jnp.float32),
            [pltpu.VMEM((lst, 128), jnp.float32) for _ in range(nbuf)],
            [pltpu.VMEM((lst,), jnp.int32) for _ in range(nbuf)],
            pltpu.VMEM((LANES,), jnp.float32),
            [pltpu.SemaphoreType.DMA for _ in range(nbuf)],
            [pltpu.SemaphoreType.DMA for _ in range(nbuf)],
        ],
        compiler_params=pltpu.CompilerParams(
            needs_layout_passes=False, disable_bounds_checks=True),
    )
    def sc_call(idx_hbm, tgt_hbm, lse_hbm, table_hbm, out_hbm,
                part_hbm, idx_v, tgt_v, lse_v, table_sh,
                img_bufs, list_bufs, acc_v, gsems, ssems):
        cid = lax.axis_index("c")
        sid = lax.axis_index("s")
        wid = sid * 2 + cid
        base = wid * per_w

        rt_per = (nrt + LANES - 1) // LANES
        for j in range(rt_per):
            rt = sid * rt_per + j

            @pl.when(rt < nrt)
            def _():
                for lt in range(nlt):
                    pltpu.sync_copy(
                        table_hbm.at[pl.ds(rt * 8, 8),
                                     pl.ds(lt * 128, 128)],
                        table_sh.at[pl.ds(lt * v + rt * 8, 8)],
                    )

        pltpu.sync_copy(idx_hbm.at[pl.ds(base, per_w)], idx_v)
        pltpu.sync_copy(tgt_hbm.at[pl.ds(base, per_w)], tgt_v)
        pltpu.sync_copy(lse_hbm, lse_v)
        plsc.subcore_barrier()

        def build_list(g, b):
            iv = idx_v[pl.ds(g * group, LANES)]
            for lt in range(nlt):
                list_bufs[b][pl.ds(lt * LANES, LANES)] = iv + lt * v

        def gather(g, b):
            return pltpu.make_async_copy(
                table_sh.at[list_bufs[b]], img_bufs[b], gsems[b])

        dyn_tail_col = (nlt - 1) * 128 + 0 * wid

        def scatters(g, b):
            slab0 = (base + g * group) // 8
            cps = []
            for lt in range(nlt):
                col = pl.ds(lt * 128, 128) if lt < nlt - 1 else pl.ds(
                    dyn_tail_col, 128)
                for h in range(group // 8):
                    cps.append(pltpu.make_async_copy(
                        img_bufs[b].at[pl.ds(lt * group + h * 8, 8)],
                        out_hbm.at[slab0 + h, :, col],
                        ssems[b]))
            return cps

        build_list(0, 0)
        gather(0, 0).start()

        def group_step(g, b, acc):
            gather(g, b).wait()
            for cp in scatters(g, b):
                cp.start()

            iv = idx_v[pl.ds(g * group, LANES)]
            tv = tgt_v[pl.ds(g * group, LANES)]
            rows = (tv // 128) * group + lax.iota(jnp.int32, LANES)
            cols = tv % 128
            tval = plsc.load_gather(img_bufs[b], [rows, cols])
            lg = plsc.load_gather(lse_v, [iv])
            acc = acc + (lg - tval)

            nb = (b + 1) % nbuf

            @pl.when(g + 1 < ngrp)
            def _():
                @pl.when(g + 1 >= nbuf)
                def _():
                    for cp in scatters(g + 1 - nbuf, nb):
                        cp.wait()

                build_list(g + 1, nb)
                gather(g + 1, nb).start()

            return acc

        def ring_body(r, acc):
            for b in range(nbuf):
                acc = group_step(r * nbuf + b, b, acc)
            return acc

        rounds = ngrp // nbuf
        acc = lax.fori_loop(
            0, rounds, ring_body, jnp.zeros((LANES,), jnp.float32)
        )
        for b in range(ngrp % nbuf):
            acc = group_step(rounds * nbuf + b, b, acc)
        for g in range(ngrp - nbuf, ngrp):
            for cp in scatters(g, g % nbuf):
                cp.wait()
        acc_v[...] = acc
        pltpu.sync_copy(acc_v, part_hbm.at[wid])

    return sc_call


def kernel(idx, targets, table):
    b, t = idx.shape
    v = table.shape[0]
    tok = b * t
    nw = 32
    nbuf = 3

    idx_f = idx.reshape(tok)
    tgt_f = targets.reshape(tok)
    lse, padt = _row_lse(table)
    out, parts = _make_sc_call(tok, v, nw, nbuf)(
        idx_f, tgt_f, lse, padt)
    logits = out.reshape(b, t, v)

    loss = jnp.sum(parts) / tok
    return (logits, loss)
